# Initial kernel scaffold; baseline (speedup 1.0000x reference)
import jax
import jax.numpy as jnp
from jax.experimental import pallas as pl


def kernel(U, a_w, a_deg, a_deg_t, b_w, b_deg, b_deg_t, src, dst, b_src, b_dst):
    raise NotImplementedError("write your pallas kernel here")


# speedup vs baseline: 6.2887x; 6.2887x over previous
import functools

import jax
import jax.numpy as jnp
from jax import lax
from jax.experimental import pallas as pl
from jax.experimental.pallas import tpu as pltpu
from jax.experimental.pallas import tpu_sc as plsc

_N = 100000
_D = 32
_C = 1000
_ZP = 0.5 / (1.0 - 0.5)
_C1 = 1.0 / 1600000.0
_C2 = 1.0 / 300000.0
_LAM = 0.5

_NC, _NS, _L = 2, 16, 16
_SUB = 128
_NSUB = 4
_CHUNK = _SUB * _NSUB


def _spmm_sc(x, srcr, dstr, wr, n_out):
    ep = srcr.shape[0] * _SUB
    ept = ep // _NS
    nchunks = ept // _CHUNK
    assert ept % _CHUNK == 0
    half = (((n_out + 1) // 2) + 127) // 128 * 128
    rpt = half // _NS
    spread_mod = max(half // _L, 1)

    mesh = plsc.VectorSubcoreMesh(
        core_axis_name="c", subcore_axis_name="s",
        num_cores=_NC, num_subcores=_NS)

    @functools.partial(
        pl.kernel, mesh=mesh,
        compiler_params=pltpu.CompilerParams(use_tc_tiling_on_sc=False),
        out_type=jax.ShapeDtypeStruct((2 * half, _D), jnp.float32),
        scratch_types=[
            pltpu.VMEM((_CHUNK, _D), jnp.float32),
            pltpu.VMEM((_NSUB, _SUB), jnp.int32),
            pltpu.VMEM((_NSUB, _SUB), jnp.int32),
            pltpu.VMEM((_NSUB, _SUB), jnp.float32),
            pltpu.VMEM((_NSUB, _SUB), jnp.int32),
            pltpu.VMEM_SHARED((half, _D), jnp.float32),
            pltpu.SemaphoreType.DMA,
        ],
    )
    def k(x_h, src_h, dst_h, w_h, out_h, rows, sv, dv, wv, dl, acc, sem):
        c = lax.axis_index("c")
        s = lax.axis_index("s")
        iota = lax.iota(jnp.int32, _L)
        zf = jnp.zeros((_L,), jnp.float32)
        zi = iota * 0

        @pl.loop(0, _CHUNK)
        def _(r):
            rows[r, 0:16] = zf
            rows[r, 16:32] = zf

        nfull, remr = rpt // _CHUNK, rpt % _CHUNK
        for p in range(nfull):
            pltpu.sync_copy(rows.at[pl.ds(0, _CHUNK)],
                            acc.at[pl.ds(s * rpt + p * _CHUNK, _CHUNK)])
        if remr:
            pltpu.sync_copy(rows.at[pl.ds(0, remr)],
                            acc.at[pl.ds(s * rpt + nfull * _CHUNK, remr)])
        plsc.subcore_barrier()

        ebase = s * (ept // _SUB)
        base_row = c * half

        @pl.loop(0, nchunks)
        def _(ch):
            crow = ebase + ch * _NSUB
            pltpu.sync_copy(src_h.at[pl.ds(crow, _NSUB)], sv)
            pltpu.sync_copy(dst_h.at[pl.ds(crow, _NSUB)], dv)
            pltpu.sync_copy(w_h.at[pl.ds(crow, _NSUB)], wv)

            @pl.loop(0, _NSUB)
            def _(j):
                pltpu.async_copy(x_h.at[sv.at[j]],
                                 rows.at[pl.ds(j * _SUB, _SUB)], sem)

            @pl.loop(0, _NSUB)
            def _(j):
                for kk in range(_SUB // _L):
                    sl = pl.ds(kk * _L, _L)
                    d16 = dv[j, sl]
                    dloc = d16 - base_row
                    own = (dloc >= 0) & (dloc < half)
                    sp = iota + lax.rem(j * (_SUB // _L) + kk, spread_mod) * _L
                    dl[j, sl] = jnp.where(own, dloc, sp)
                    wv[j, sl] = jnp.where(own, wv[j, sl], 0.0)

            @pl.loop(0, _NSUB)
            def _(j):
                pltpu.make_async_copy(
                    x_h.at[sv.at[0]], rows.at[pl.ds(0, _SUB)], sem).wait()

            @pl.loop(0, _NSUB)
            def _(j):
                @pl.loop(0, _SUB // _L)
                def _(kk):
                    w16 = wv[j, pl.ds(kk * _L, _L)]
                    base = j * _SUB + kk * _L
                    for m in range(_L):
                        w_s = w16[m]
                        r = base + m
                        rows[r, 0:16] = rows[r, 0:16] * w_s
                        rows[r, 16:32] = rows[r, 16:32] * w_s
                pltpu.sync_copy(rows.at[pl.ds(j * _SUB, _SUB)],
                                acc.at[dl.at[j]], add=True)

        plsc.subcore_barrier()

        off = 0
        for p in range(rpt // _CHUNK):
            pltpu.sync_copy(acc.at[pl.ds(s * rpt + off, _CHUNK)],
                            out_h.at[pl.ds(base_row + s * rpt + off, _CHUNK)])
            off += _CHUNK
        if rpt % _CHUNK:
            rem = rpt % _CHUNK
            pltpu.sync_copy(acc.at[pl.ds(s * rpt + off, rem)],
                            out_h.at[pl.ds(base_row + s * rpt + off, rem)])

    return k(x, srcr, dstr, wr)[:n_out]


def _pad_edges(src, dst, w, n_in, n_out):
    e = src.shape[0]
    ep = -(-e // (_NS * _CHUNK)) * (_NS * _CHUNK)
    pad = ep - e
    if pad:
        r = jnp.arange(pad, dtype=jnp.int32)
        src = jnp.concatenate([src, r % n_in])
        dst = jnp.concatenate([dst, r % n_out])
        w = jnp.concatenate([w, jnp.zeros((pad,), jnp.float32)])
    return (src.reshape(ep // _SUB, _SUB), dst.reshape(ep // _SUB, _SUB),
            w.reshape(ep // _SUB, _SUB))


_BLK = 4000
_NBLK = _N // _BLK


def _dot_t(a, b):
    return lax.dot_general(a, b, (((0,), (0,)), ((), ())),
                           precision=lax.Precision.HIGHEST,
                           preferred_element_type=jnp.float32)


def _reduce2(X, va, vb):
    def body(x_ref, va_ref, vb_ref, sa_ref, sb_ref):
        i = pl.program_id(0)

        @pl.when(i == 0)
        def _():
            sa_ref[...] = jnp.zeros_like(sa_ref)
            sb_ref[...] = jnp.zeros_like(sb_ref)
        x = x_ref[...]
        sa_ref[...] += jnp.sum(va_ref[...] * x, axis=0, keepdims=True)
        sb_ref[...] += jnp.sum(vb_ref[...] * x, axis=0, keepdims=True)

    return pl.pallas_call(
        body,
        grid=(_NBLK,),
        in_specs=[pl.BlockSpec((_BLK, _D), lambda i: (i, 0)),
                  pl.BlockSpec((_BLK, 1), lambda i: (i, 0)),
                  pl.BlockSpec((_BLK, 1), lambda i: (i, 0))],
        out_specs=[pl.BlockSpec((1, _D), lambda i: (0, 0)),
                   pl.BlockSpec((1, _D), lambda i: (0, 0))],
        out_shape=[jax.ShapeDtypeStruct((1, _D), jnp.float32)] * 2,
    )(X, va, vb)


def _layer_step(h, a_deg, adt, sprev, Z):
    def body(h_ref, ad_ref, adt_ref, s_ref, z_ref, lz_ref, zn_ref, sn_ref):
        i = pl.program_id(0)

        @pl.when(i == 0)
        def _():
            sn_ref[...] = jnp.zeros_like(sn_ref)
        lz = _ZP * (h_ref[...] - _C1 * ad_ref[...] * s_ref[...])
        lz_ref[...] = lz
        zn_ref[...] = z_ref[...] + lz
        sn_ref[...] += jnp.sum(adt_ref[...] * lz, axis=0, keepdims=True)

    return pl.pallas_call(
        body,
        grid=(_NBLK,),
        in_specs=[pl.BlockSpec((_BLK, _D), lambda i: (i, 0)),
                  pl.BlockSpec((_BLK, 1), lambda i: (i, 0)),
                  pl.BlockSpec((_BLK, 1), lambda i: (i, 0)),
                  pl.BlockSpec((1, _D), lambda i: (0, 0)),
                  pl.BlockSpec((_BLK, _D), lambda i: (i, 0))],
        out_specs=[pl.BlockSpec((_BLK, _D), lambda i: (i, 0)),
                   pl.BlockSpec((_BLK, _D), lambda i: (i, 0)),
                   pl.BlockSpec((1, _D), lambda i: (0, 0))],
        out_shape=[jax.ShapeDtypeStruct((_N, _D), jnp.float32),
                   jax.ShapeDtypeStruct((_N, _D), jnp.float32),
                   jax.ShapeDtypeStruct((1, _D), jnp.float32)],
    )(h, a_deg, adt, sprev, Z)


def _comm1(bc, bdt, sb):
    def body(bc_ref, bdt_ref, sb_ref, tr_ref, s2_ref):
        tr = bc_ref[...] - _C2 * bdt_ref[...] * sb_ref[...]
        tr_ref[...] = tr
        s2_ref[...] = jnp.sum(bdt_ref[...] * tr, axis=0, keepdims=True)

    return pl.pallas_call(
        body,
        out_shape=[jax.ShapeDtypeStruct((_C, _D), jnp.float32),
                   jax.ShapeDtypeStruct((1, _D), jnp.float32)],
    )(bc, bdt, sb)


def _comm2(bc2, b_deg, s2):
    def body(b_ref, bd_ref, s_ref, r_ref):
        r_ref[...] = b_ref[...] - _C2 * bd_ref[...] * s_ref[...]

    return pl.pallas_call(
        body,
        grid=(_NBLK,),
        in_specs=[pl.BlockSpec((_BLK, _D), lambda i: (i, 0)),
                  pl.BlockSpec((_BLK, 1), lambda i: (i, 0)),
                  pl.BlockSpec((1, _D), lambda i: (0, 0))],
        out_specs=pl.BlockSpec((_BLK, _D), lambda i: (i, 0)),
        out_shape=jax.ShapeDtypeStruct((_N, _D), jnp.float32),
    )(bc2, b_deg, s2)


def _rownorm(x):
    n = jnp.sqrt(jnp.sum(x * x, axis=1, keepdims=True))
    return x / jnp.maximum(n, 1e-12)


def _hsic_reduce(dZ, dR):
    def body(z_ref, r_ref, gz_ref, gr_ref):
        i = pl.program_id(0)

        @pl.when(i == 0)
        def _():
            gz_ref[...] = jnp.zeros_like(gz_ref)
            gr_ref[...] = jnp.zeros_like(gr_ref)
        z = z_ref[...]
        r = r_ref[...]
        gz_ref[...] += _dot_t(_rownorm(r), z)
        gr_ref[...] += _dot_t(_rownorm(z), r)

    return pl.pallas_call(
        body,
        grid=(_NBLK,),
        in_specs=[pl.BlockSpec((_BLK, _D), lambda i: (i, 0)),
                  pl.BlockSpec((_BLK, _D), lambda i: (i, 0))],
        out_specs=[pl.BlockSpec((_D, _D), lambda i: (0, 0)),
                   pl.BlockSpec((_D, _D), lambda i: (0, 0))],
        out_shape=[jax.ShapeDtypeStruct((_D, _D), jnp.float32)] * 2,
    )(dZ, dR)


def _hsic_apply(dZ, dR, Gz, Gr):
    def body(z_ref, r_ref, gz_ref, gr_ref, zo_ref, ro_ref):
        z = z_ref[...]
        r = r_ref[...]
        zo_ref[...] = z - _LAM * jnp.dot(
            _rownorm(r), gz_ref[...], precision=lax.Precision.HIGHEST,
            preferred_element_type=jnp.float32)
        ro_ref[...] = r - _LAM * jnp.dot(
            _rownorm(z), gr_ref[...], precision=lax.Precision.HIGHEST,
            preferred_element_type=jnp.float32)

    return pl.pallas_call(
        body,
        grid=(_NBLK,),
        in_specs=[pl.BlockSpec((_BLK, _D), lambda i: (i, 0)),
                  pl.BlockSpec((_BLK, _D), lambda i: (i, 0)),
                  pl.BlockSpec((_D, _D), lambda i: (0, 0)),
                  pl.BlockSpec((_D, _D), lambda i: (0, 0))],
        out_specs=[pl.BlockSpec((_BLK, _D), lambda i: (i, 0)),
                   pl.BlockSpec((_BLK, _D), lambda i: (i, 0))],
        out_shape=[jax.ShapeDtypeStruct((_N, _D), jnp.float32)] * 2,
    )(dZ, dR, Gz, Gr)


def kernel(U, a_w, a_deg, a_deg_t, b_w, b_deg, b_deg_t, src, dst, b_src, b_dst):
    adt = a_deg_t.reshape(_N, 1)
    bdt = b_deg_t.reshape(_C, 1)

    sa, sb = _reduce2(U, adt, b_deg)

    srcA, dstA, wA = _pad_edges(src, dst, a_w, _N, _N)
    srcB1, dstB1, wB1 = _pad_edges(b_src, b_dst, b_w, _N, _C)
    srcB2, dstB2, wB2 = _pad_edges(b_dst, b_src, b_w, _C, _N)

    h1 = _spmm_sc(U, srcA, dstA, wA, _N)
    lz1, Zc, s1 = _layer_step(h1, a_deg, adt, sa, U)
    h2 = _spmm_sc(lz1, srcA, dstA, wA, _N)
    lz2, Zc, s2 = _layer_step(h2, a_deg, adt, s1, Zc)
    h3 = _spmm_sc(lz2, srcA, dstA, wA, _N)
    _, Z1, _ = _layer_step(h3, a_deg, adt, s2, Zc)
    Z2 = _spmm_sc(h1, srcA, dstA, wA, _N)

    bc = _spmm_sc(U, srcB1, dstB1, wB1, _C)
    tmp_R, s2b = _comm1(bc, bdt, sb)
    bc2 = _spmm_sc(tmp_R, srcB2, dstB2, wB2, _N)
    R = _comm2(bc2, b_deg, s2b)

    dZ, dR = Z1, R
    for _ in range(2):
        Gz, Gr = _hsic_reduce(dZ, dR)
        dZ, dR = _hsic_apply(dZ, dR, Gz, Gr)

    return jnp.stack([dZ, dR, Z2], axis=0)
